# baseline (speedup 1.0000x reference)
import functools

import jax
import jax.numpy as jnp
from jax import lax
from jax.experimental import pallas as pl
from jax.experimental.pallas import tpu as pltpu
from jax.experimental.pallas import tpu_sc as plsc

_N = 10000
_D = 128
_E = 320000
_NC = 2
_NS = 16
_NW = _NC * _NS
_EPW = _E // _NW
_CH = 80
_NCHUNK = _EPW // _CH
_NSTAGE = 25
_CPS = _NCHUNK // _NSTAGE
_RPT = _N // _NS
_BLK = 2000
_GRID = _N // _BLK


def _mm3_body(x_ref, w1_ref, b1_ref, w2_ref, w3_ref, b3_ref,
              a_ref, c_ref, d_ref):
    x = x_ref[...]
    a_ref[...] = jnp.dot(x, w1_ref[...], preferred_element_type=jnp.float32) + b1_ref[...]
    c_ref[...] = jnp.dot(x, w2_ref[...], preferred_element_type=jnp.float32)
    d_ref[...] = jnp.dot(x, w3_ref[...], preferred_element_type=jnp.float32) + b3_ref[...]


_W_SPEC = pl.BlockSpec((_D, _D), lambda i: (0, 0))
_B_SPEC = pl.BlockSpec((1, _D), lambda i: (0, 0))
_X_SPEC = pl.BlockSpec((_BLK, _D), lambda i: (i, 0))
_P0_SPEC = pl.BlockSpec((_BLK, _D), lambda i: (i, 0))
_P1_SPEC = pl.BlockSpec((_BLK, _D), lambda i: (i + _GRID, 0))

_mm3 = pl.pallas_call(
    _mm3_body,
    grid=(_GRID,),
    in_specs=[_X_SPEC, _W_SPEC, _B_SPEC, _W_SPEC, _W_SPEC, _B_SPEC],
    out_specs=[_X_SPEC, _X_SPEC, _X_SPEC],
    out_shape=[jax.ShapeDtypeStruct((_N, _D), jnp.float32)] * 3,
)


def _comb_mm3_body(p0_ref, p1_ref, d_ref,
                   w1_ref, b1_ref, w2_ref, w3_ref, b3_ref,
                   a_ref, c1_ref, d1_ref):
    h = p0_ref[...] + p1_ref[...] + d_ref[...]
    h = jnp.where(h >= 0, h, 0.01 * h)
    a_ref[...] = jnp.dot(h, w1_ref[...], preferred_element_type=jnp.float32) + b1_ref[...]
    c1_ref[...] = jnp.dot(h, w2_ref[...], preferred_element_type=jnp.float32)
    d1_ref[...] = jnp.dot(h, w3_ref[...], preferred_element_type=jnp.float32) + b3_ref[...]


_comb_mm3 = pl.pallas_call(
    _comb_mm3_body,
    grid=(_GRID,),
    in_specs=[_P0_SPEC, _P1_SPEC, _X_SPEC,
              _W_SPEC, _B_SPEC, _W_SPEC, _W_SPEC, _B_SPEC],
    out_specs=[_X_SPEC, _X_SPEC, _X_SPEC],
    out_shape=[jax.ShapeDtypeStruct((_N, _D), jnp.float32)] * 3,
)


def _comb_body(p0_ref, p1_ref, d_ref, o_ref):
    h = p0_ref[...] + p1_ref[...] + d_ref[...]
    o_ref[...] = jnp.where(h >= 0, h, 0.01 * h)


_comb = pl.pallas_call(
    _comb_body,
    grid=(_GRID,),
    in_specs=[_P0_SPEC, _P1_SPEC, _X_SPEC],
    out_specs=pl.BlockSpec((_BLK, _D), lambda i: (i, 0)),
    out_shape=jax.ShapeDtypeStruct((_N, _D), jnp.float32),
)


def _sc_body(a_hbm, b_hbm, src_hbm, dst_hbm, ew_hbm, acc_out,
             src_v, dst_v, ew_v, ra0, ra1, rb0, rb1, acc_s,
             sem_a0, sem_a1, sem_b0, sem_b1, sem_s0, sem_s1, sem_i):
    cid = lax.axis_index("c")
    sid = lax.axis_index("s")
    wid = sid * _NC + cid

    zeros16 = jnp.zeros((16,), jnp.float32)
    ra = (ra0, ra1)
    rb = (rb0, rb1)
    sem_a = (sem_a0, sem_a1)
    sem_b = (sem_b0, sem_b1)
    sem_s = (sem_s0, sem_s1)
    n_total = _NCHUNK

    def zrow(r, carry):
        for k in range(8):
            ra0[r, pl.ds(k * 16, 16)] = zeros16
        return carry
    lax.fori_loop(jnp.int32(0), jnp.int32(_CH), zrow, jnp.int32(0))

    def zcopy(q, carry):
        pltpu.sync_copy(ra0, acc_s.at[pl.ds(sid * _RPT + q * _CH, _CH)])
        return carry
    lax.fori_loop(jnp.int32(0), jnp.int32(7), zcopy, jnp.int32(0))
    pltpu.sync_copy(ra0.at[pl.ds(0, 65)],
                    acc_s.at[pl.ds(sid * _RPT + 560, 65)])

    plsc.subcore_barrier()

    pltpu.sync_copy(src_hbm.at[wid, jnp.int32(0)], src_v.at[jnp.int32(0)])
    pltpu.sync_copy(dst_hbm.at[wid, jnp.int32(0)], dst_v.at[jnp.int32(0)])
    pltpu.sync_copy(ew_hbm.at[wid, jnp.int32(0)], ew_v.at[jnp.int32(0)])
    z32 = jnp.int32(0)
    pltpu.async_copy(a_hbm.at[src_v.at[z32, z32]], ra[0], sem_a[0])
    pltpu.async_copy(b_hbm.at[dst_v.at[z32, z32]], rb[0], sem_b[0])

    def chunk_body(j, P):
        Q = 1 - P
        st = j // jnp.int32(_CPS)
        jj = lax.rem(j, jnp.int32(_CPS))
        sl = lax.rem(st, jnp.int32(2))
        j1 = j + jnp.int32(1)
        st1 = j1 // jnp.int32(_CPS)
        jj1 = lax.rem(j1, jnp.int32(_CPS))
        sl1 = lax.rem(st1, jnp.int32(2))

        pltpu.make_async_copy(a_hbm.at[src_v.at[z32, z32]], ra[P], sem_a[P]).wait()
        pltpu.make_async_copy(b_hbm.at[dst_v.at[z32, z32]], rb[P], sem_b[P]).wait()

        rap = ra[P]
        rbp = rb[P]

        def group(g, c2):
            @pl.when(jnp.logical_and(g == 1, j < n_total - 1))
            def _():
                pltpu.async_copy(b_hbm.at[dst_v.at[sl1, jj1]], rb[Q], sem_b[Q])

            @pl.when(jnp.logical_and(g == 2, j >= 1))
            def _():
                pltpu.make_async_copy(ra[Q], acc_s.at[dst_v.at[z32, z32]],
                                      sem_s[Q]).wait()

            @pl.when(jnp.logical_and(g == 2, j < n_total - 1))
            def _():
                pltpu.async_copy(a_hbm.at[src_v.at[sl1, jj1]], ra[Q], sem_a[Q])

            ewg = ew_v[sl, jj, pl.ds(g * 16, 16)]
            base = g * jnp.int32(16)
            for t in range(16):
                w = ewg[t]
                wv = jnp.full((16,), w, jnp.float32)
                e = base + t
                for k in range(8):
                    slc = pl.ds(k * 16, 16)
                    rap[e, slc] = (rap[e, slc] - rbp[e, slc]) * wv
            return c2
        lax.fori_loop(jnp.int32(0), jnp.int32(_CH // 16), group, jnp.int32(0))

        pltpu.async_copy(rap, acc_s.at[dst_v.at[sl, jj]], sem_s[P], add=True)

    def chunk(j, carry):
        st = j // jnp.int32(_CPS)
        jj = lax.rem(j, jnp.int32(_CPS))
        sl = lax.rem(st, jnp.int32(2))
        nsl = jnp.int32(1) - sl

        @pl.when(jnp.logical_and(jj == 0, st < _NSTAGE - 1))
        def _():
            st1 = st + jnp.int32(1)
            pltpu.async_copy(src_hbm.at[wid, st1], src_v.at[nsl], sem_i)
            pltpu.async_copy(dst_hbm.at[wid, st1], dst_v.at[nsl], sem_i)
            pltpu.async_copy(ew_hbm.at[wid, st1], ew_v.at[nsl], sem_i)

        @pl.when(jnp.logical_and(jj == _CPS - 1, st < _NSTAGE - 1))
        def _():
            pltpu.make_async_copy(src_hbm.at[wid, st], src_v.at[sl], sem_i).wait()
            pltpu.make_async_copy(dst_hbm.at[wid, st], dst_v.at[sl], sem_i).wait()
            pltpu.make_async_copy(ew_hbm.at[wid, st], ew_v.at[sl], sem_i).wait()

        @pl.when(lax.rem(j, jnp.int32(2)) == 0)
        def _():
            chunk_body(j, 0)

        @pl.when(lax.rem(j, jnp.int32(2)) == 1)
        def _():
            chunk_body(j, 1)
        return carry
    lax.fori_loop(jnp.int32(0), jnp.int32(n_total), chunk, jnp.int32(0))

    pltpu.make_async_copy(ra[0], acc_s.at[dst_v.at[z32, z32]], sem_s[0]).wait()

    plsc.subcore_barrier()

    wbase = pl.multiple_of(sid * jnp.int32(624), 16)
    obase = pl.multiple_of(cid * jnp.int32(_N) + sid * jnp.int32(624), 16)

    def wcopy(q, carry):
        off = pl.multiple_of(q * jnp.int32(16), 16)
        pltpu.sync_copy(acc_s.at[pl.ds(wbase + off, 16)],
                        acc_out.at[pl.ds(obase + off, 16)])
        return carry
    lax.fori_loop(jnp.int32(0), jnp.int32(39), wcopy, jnp.int32(0))

    @pl.when(sid == _NS - 1)
    def _():
        off = pl.multiple_of(jnp.int32(39 * 16), 16)
        pltpu.sync_copy(acc_s.at[pl.ds(wbase + off, 16)],
                        acc_out.at[pl.ds(obase + off, 16)])


@functools.lru_cache(maxsize=1)
def _make_sc():
    mesh = plsc.VectorSubcoreMesh(core_axis_name="c", subcore_axis_name="s")
    out_type = jax.ShapeDtypeStruct((2 * _N, _D), jnp.float32)
    scratch = [
        pltpu.VMEM((2, _CPS, _CH), jnp.int32),
        pltpu.VMEM((2, _CPS, _CH), jnp.int32),
        pltpu.VMEM((2, _CPS, _CH), jnp.float32),
        pltpu.VMEM((_CH, _D), jnp.float32),
        pltpu.VMEM((_CH, _D), jnp.float32),
        pltpu.VMEM((_CH, _D), jnp.float32),
        pltpu.VMEM((_CH, _D), jnp.float32),
        pltpu.VMEM_SHARED((_N, _D), jnp.float32),
        pltpu.SemaphoreType.DMA,
        pltpu.SemaphoreType.DMA,
        pltpu.SemaphoreType.DMA,
        pltpu.SemaphoreType.DMA,
        pltpu.SemaphoreType.DMA,
        pltpu.SemaphoreType.DMA,
        pltpu.SemaphoreType.DMA,
    ]
    return pl.kernel(_sc_body, out_type=out_type, mesh=mesh,
                     scratch_types=scratch)


def kernel(y, edge_index, edge_weight,
           W1_0, b1_0, W2_0, W3_0, b3_0,
           W1_1, b1_1, W2_1, W3_1, b3_1):
    with jax.enable_x64(False):
        return _kernel32(y, edge_index, edge_weight,
                         W1_0, b1_0, W2_0, W3_0, b3_0,
                         W1_1, b1_1, W2_1, W3_1, b3_1)


def _kernel32(y, edge_index, edge_weight,
              W1_0, b1_0, W2_0, W3_0, b3_0,
              W1_1, b1_1, W2_1, W3_1, b3_1):
    y = y.astype(jnp.float32)
    src = edge_index[0].astype(jnp.int32).reshape(_NW, _NSTAGE, _CPS, _CH)
    dst = edge_index[1].astype(jnp.int32).reshape(_NW, _NSTAGE, _CPS, _CH)
    ew = edge_weight.astype(jnp.float32).reshape(_NW, _NSTAGE, _CPS, _CH)
    b1_0r = b1_0.reshape(1, _D).astype(jnp.float32)
    b3_0r = b3_0.reshape(1, _D).astype(jnp.float32)
    b1_1r = b1_1.reshape(1, _D).astype(jnp.float32)
    b3_1r = b3_1.reshape(1, _D).astype(jnp.float32)

    sc = _make_sc()

    a0, c0, d0 = _mm3(y, W1_0, b1_0r, W2_0, W3_0, b3_0r)
    p0 = sc(a0, c0, src, dst, ew)
    a1, c1, d1 = _comb_mm3(p0, p0, d0, W1_1, b1_1r, W2_1, W3_1, b3_1r)
    p1 = sc(a1, c1, src, dst, ew)
    return _comb(p1, p1, d1)


# speedup vs baseline: 1.0745x; 1.0745x over previous
import functools

import jax
import jax.numpy as jnp
from jax import lax
from jax.experimental import pallas as pl
from jax.experimental.pallas import tpu as pltpu
from jax.experimental.pallas import tpu_sc as plsc

_N = 10000
_D = 128
_E = 320000
_NC = 2
_NS = 16
_NW = _NC * _NS
_EPW = _E // _NW
_CH = 80
_NCHUNK = _EPW // _CH
_NSTAGE = 25
_CPS = _NCHUNK // _NSTAGE
_RPT = _N // _NS
_BLK = 2000
_GRID = _N // _BLK


def _mm3_body(x_ref, w1_ref, b1_ref, w2_ref, w3_ref, b3_ref,
              a_ref, c_ref, d_ref):
    x = x_ref[...]
    a_ref[...] = jnp.dot(x, w1_ref[...], preferred_element_type=jnp.float32) + b1_ref[...]
    c_ref[...] = jnp.dot(x, w2_ref[...], preferred_element_type=jnp.float32)
    d_ref[...] = jnp.dot(x, w3_ref[...], preferred_element_type=jnp.float32) + b3_ref[...]


_W_SPEC = pl.BlockSpec((_D, _D), lambda i: (0, 0))
_B_SPEC = pl.BlockSpec((1, _D), lambda i: (0, 0))
_X_SPEC = pl.BlockSpec((_BLK, _D), lambda i: (i, 0))
_P0_SPEC = pl.BlockSpec((_BLK, _D), lambda i: (i, 0))
_P1_SPEC = pl.BlockSpec((_BLK, _D), lambda i: (i + _GRID, 0))

_mm3 = pl.pallas_call(
    _mm3_body,
    grid=(_GRID,),
    in_specs=[_X_SPEC, _W_SPEC, _B_SPEC, _W_SPEC, _W_SPEC, _B_SPEC],
    out_specs=[_X_SPEC, _X_SPEC, _X_SPEC],
    out_shape=[jax.ShapeDtypeStruct((_N, _D), jnp.float32)] * 3,
)


def _comb_mm3_body(p0_ref, p1_ref, d_ref,
                   w1_ref, b1_ref, w2_ref, w3_ref, b3_ref,
                   a_ref, c1_ref, d1_ref):
    h = p0_ref[...] + p1_ref[...] + d_ref[...]
    h = jnp.where(h >= 0, h, 0.01 * h)
    a_ref[...] = jnp.dot(h, w1_ref[...], preferred_element_type=jnp.float32) + b1_ref[...]
    c1_ref[...] = jnp.dot(h, w2_ref[...], preferred_element_type=jnp.float32)
    d1_ref[...] = jnp.dot(h, w3_ref[...], preferred_element_type=jnp.float32) + b3_ref[...]


_comb_mm3 = pl.pallas_call(
    _comb_mm3_body,
    grid=(_GRID,),
    in_specs=[_P0_SPEC, _P1_SPEC, _X_SPEC,
              _W_SPEC, _B_SPEC, _W_SPEC, _W_SPEC, _B_SPEC],
    out_specs=[_X_SPEC, _X_SPEC, _X_SPEC],
    out_shape=[jax.ShapeDtypeStruct((_N, _D), jnp.float32)] * 3,
)


def _comb_body(p0_ref, p1_ref, d_ref, o_ref):
    h = p0_ref[...] + p1_ref[...] + d_ref[...]
    o_ref[...] = jnp.where(h >= 0, h, 0.01 * h)


_comb = pl.pallas_call(
    _comb_body,
    grid=(_GRID,),
    in_specs=[_P0_SPEC, _P1_SPEC, _X_SPEC],
    out_specs=pl.BlockSpec((_BLK, _D), lambda i: (i, 0)),
    out_shape=jax.ShapeDtypeStruct((_N, _D), jnp.float32),
)


def _sc_body(a_hbm, b_hbm, src_hbm, dst_hbm, ew_hbm, acc_out,
             src_v, dst_v, ew_v, ra0, ra1, rb0, rb1, acc_s,
             sem_a0, sem_a1, sem_b0, sem_b1, sem_s0, sem_s1, sem_i):
    cid = lax.axis_index("c")
    sid = lax.axis_index("s")
    wid = sid * _NC + cid

    zeros16 = jnp.zeros((16,), jnp.float32)
    ra = (ra0, ra1)
    rb = (rb0, rb1)
    sem_a = (sem_a0, sem_a1)
    sem_b = (sem_b0, sem_b1)
    sem_s = (sem_s0, sem_s1)
    n_total = _NCHUNK

    def zrow(r, carry):
        for k in range(8):
            ra0[r, pl.ds(k * 16, 16)] = zeros16
        return carry
    lax.fori_loop(jnp.int32(0), jnp.int32(_CH), zrow, jnp.int32(0))

    def zcopy(q, carry):
        pltpu.sync_copy(ra0, acc_s.at[pl.ds(sid * _RPT + q * _CH, _CH)])
        return carry
    lax.fori_loop(jnp.int32(0), jnp.int32(7), zcopy, jnp.int32(0))
    pltpu.sync_copy(ra0.at[pl.ds(0, 65)],
                    acc_s.at[pl.ds(sid * _RPT + 560, 65)])

    plsc.subcore_barrier()

    pltpu.sync_copy(src_hbm.at[wid, jnp.int32(0)], src_v.at[jnp.int32(0)])
    pltpu.sync_copy(dst_hbm.at[wid, jnp.int32(0)], dst_v.at[jnp.int32(0)])
    pltpu.sync_copy(ew_hbm.at[wid, jnp.int32(0)], ew_v.at[jnp.int32(0)])
    z32 = jnp.int32(0)
    pltpu.async_copy(a_hbm.at[src_v.at[z32, z32]], ra[0], sem_a[0])
    pltpu.async_copy(b_hbm.at[dst_v.at[z32, z32]], rb[0], sem_b[0])

    def chunk_body(j, P):
        Q = 1 - P
        st = j // jnp.int32(_CPS)
        jj = lax.rem(j, jnp.int32(_CPS))
        sl = lax.rem(st, jnp.int32(2))
        j1 = j + jnp.int32(1)
        st1 = j1 // jnp.int32(_CPS)
        jj1 = lax.rem(j1, jnp.int32(_CPS))
        sl1 = lax.rem(st1, jnp.int32(2))

        pltpu.make_async_copy(a_hbm.at[src_v.at[z32, z32]], ra[P], sem_a[P]).wait()
        pltpu.make_async_copy(b_hbm.at[dst_v.at[z32, z32]], rb[P], sem_b[P]).wait()

        rap = ra[P]
        rbp = rb[P]

        def group(g, c2):
            @pl.when(jnp.logical_and(g == 1, j >= 1))
            def _():
                pltpu.make_async_copy(ra[Q], acc_s.at[dst_v.at[z32, z32]],
                                      sem_s[Q]).wait()

            @pl.when(jnp.logical_and(g == 1, j < n_total - 1))
            def _():
                pltpu.async_copy(a_hbm.at[src_v.at[sl1, jj1]], ra[Q], sem_a[Q])
                pltpu.async_copy(b_hbm.at[dst_v.at[sl1, jj1]], rb[Q], sem_b[Q])

            ewg = ew_v[sl, jj, pl.ds(g * 16, 16)]
            base = g * jnp.int32(16)
            for t in range(16):
                w = ewg[t]
                wv = jnp.full((16,), w, jnp.float32)
                e = base + t
                for k in range(8):
                    slc = pl.ds(k * 16, 16)
                    rap[e, slc] = (rap[e, slc] - rbp[e, slc]) * wv
            return c2
        lax.fori_loop(jnp.int32(0), jnp.int32(_CH // 16), group, jnp.int32(0))

        pltpu.async_copy(rap, acc_s.at[dst_v.at[sl, jj]], sem_s[P], add=True)

    def chunk(j, carry):
        st = j // jnp.int32(_CPS)
        jj = lax.rem(j, jnp.int32(_CPS))
        sl = lax.rem(st, jnp.int32(2))
        nsl = jnp.int32(1) - sl

        @pl.when(jnp.logical_and(jj == 0, st < _NSTAGE - 1))
        def _():
            st1 = st + jnp.int32(1)
            pltpu.async_copy(src_hbm.at[wid, st1], src_v.at[nsl], sem_i)
            pltpu.async_copy(dst_hbm.at[wid, st1], dst_v.at[nsl], sem_i)
            pltpu.async_copy(ew_hbm.at[wid, st1], ew_v.at[nsl], sem_i)

        @pl.when(jnp.logical_and(jj == _CPS - 1, st < _NSTAGE - 1))
        def _():
            pltpu.make_async_copy(src_hbm.at[wid, st], src_v.at[sl], sem_i).wait()
            pltpu.make_async_copy(dst_hbm.at[wid, st], dst_v.at[sl], sem_i).wait()
            pltpu.make_async_copy(ew_hbm.at[wid, st], ew_v.at[sl], sem_i).wait()

        @pl.when(lax.rem(j, jnp.int32(2)) == 0)
        def _():
            chunk_body(j, 0)

        @pl.when(lax.rem(j, jnp.int32(2)) == 1)
        def _():
            chunk_body(j, 1)
        return carry
    lax.fori_loop(jnp.int32(0), jnp.int32(n_total), chunk, jnp.int32(0))

    pltpu.make_async_copy(ra[0], acc_s.at[dst_v.at[z32, z32]], sem_s[0]).wait()

    plsc.subcore_barrier()

    wbase = pl.multiple_of(sid * jnp.int32(624), 16)
    obase = pl.multiple_of(cid * jnp.int32(_N) + sid * jnp.int32(624), 16)

    def wcopy(q, carry):
        off = pl.multiple_of(q * jnp.int32(16), 16)
        pltpu.sync_copy(acc_s.at[pl.ds(wbase + off, 16)],
                        acc_out.at[pl.ds(obase + off, 16)])
        return carry
    lax.fori_loop(jnp.int32(0), jnp.int32(39), wcopy, jnp.int32(0))

    @pl.when(sid == _NS - 1)
    def _():
        off = pl.multiple_of(jnp.int32(39 * 16), 16)
        pltpu.sync_copy(acc_s.at[pl.ds(wbase + off, 16)],
                        acc_out.at[pl.ds(obase + off, 16)])


@functools.lru_cache(maxsize=1)
def _make_sc():
    mesh = plsc.VectorSubcoreMesh(core_axis_name="c", subcore_axis_name="s")
    out_type = jax.ShapeDtypeStruct((2 * _N, _D), jnp.float32)
    scratch = [
        pltpu.VMEM((2, _CPS, _CH), jnp.int32),
        pltpu.VMEM((2, _CPS, _CH), jnp.int32),
        pltpu.VMEM((2, _CPS, _CH), jnp.float32),
        pltpu.VMEM((_CH, _D), jnp.float32),
        pltpu.VMEM((_CH, _D), jnp.float32),
        pltpu.VMEM((_CH, _D), jnp.float32),
        pltpu.VMEM((_CH, _D), jnp.float32),
        pltpu.VMEM_SHARED((_N, _D), jnp.float32),
        pltpu.SemaphoreType.DMA,
        pltpu.SemaphoreType.DMA,
        pltpu.SemaphoreType.DMA,
        pltpu.SemaphoreType.DMA,
        pltpu.SemaphoreType.DMA,
        pltpu.SemaphoreType.DMA,
        pltpu.SemaphoreType.DMA,
    ]
    return pl.kernel(_sc_body, out_type=out_type, mesh=mesh,
                     scratch_types=scratch)


def kernel(y, edge_index, edge_weight,
           W1_0, b1_0, W2_0, W3_0, b3_0,
           W1_1, b1_1, W2_1, W3_1, b3_1):
    with jax.enable_x64(False):
        return _kernel32(y, edge_index, edge_weight,
                         W1_0, b1_0, W2_0, W3_0, b3_0,
                         W1_1, b1_1, W2_1, W3_1, b3_1)


def _kernel32(y, edge_index, edge_weight,
              W1_0, b1_0, W2_0, W3_0, b3_0,
              W1_1, b1_1, W2_1, W3_1, b3_1):
    y = y.astype(jnp.float32)
    src = edge_index[0].astype(jnp.int32).reshape(_NW, _NSTAGE, _CPS, _CH)
    dst = edge_index[1].astype(jnp.int32).reshape(_NW, _NSTAGE, _CPS, _CH)
    ew = edge_weight.astype(jnp.float32).reshape(_NW, _NSTAGE, _CPS, _CH)
    b1_0r = b1_0.reshape(1, _D).astype(jnp.float32)
    b3_0r = b3_0.reshape(1, _D).astype(jnp.float32)
    b1_1r = b1_1.reshape(1, _D).astype(jnp.float32)
    b3_1r = b3_1.reshape(1, _D).astype(jnp.float32)

    sc = _make_sc()

    a0, c0, d0 = _mm3(y, W1_0, b1_0r, W2_0, W3_0, b3_0r)
    p0 = sc(a0, c0, src, dst, ew)
    a1, c1, d1 = _comb_mm3(p0, p0, d0, W1_1, b1_1r, W2_1, W3_1, b3_1r)
    p1 = sc(a1, c1, src, dst, ew)
    return _comb(p1, p1, d1)
